# baseline (speedup 1.0000x reference)
import jax
import jax.numpy as jnp
from jax.experimental import pallas as pl
from jax.experimental.pallas import tpu as pltpu

D_MODEL_ = 1024
D_INNER_ = 2048
N_STATE_ = 16
DT_RANK_ = 64
T1 = 512
NB = 1024
T2 = 256
DB = 2048
T3 = 512


def _silu(v):
    return v * jax.nn.sigmoid(v)


def _softplus(v):
    return jnp.maximum(v, 0.0) + jnp.log1p(jnp.exp(-jnp.abs(v)))


def _k1(x_ref, w_ref, cw_ref, cb_ref, xp_ref, dtw_ref, dtb_ref,
        xcz_ref, dlt_ref, bc_ref, tail_ref, acc_ref):
    l = pl.program_id(1)
    n = pl.program_id(2)
    H = T1 // 2

    def _taps(ext):
        return (ext[0:H] * cw_ref[:, 0] + ext[1:H + 1] * cw_ref[:, 1]
                + ext[2:H + 2] * cw_ref[:, 2] + ext[3:H + 3] * cw_ref[:, 3]
                + cb_ref[...])

    @pl.when(n < 2)
    def _conv_half():
        xh = x_ref[0].astype(jnp.bfloat16)
        xz1 = jnp.dot(xh[:H], w_ref[...], preferred_element_type=jnp.float32)
        xz2 = jnp.dot(xh[H:], w_ref[...], preferred_element_type=jnp.float32)
        prev = tail_ref[n, 5:8, :]
        prev = jnp.where(l == 0, jnp.zeros_like(prev), prev)
        s1 = _silu(_taps(jnp.concatenate([prev, xz1], axis=0)))
        s2 = _silu(_taps(jnp.concatenate([xz1[H - 3:], xz2], axis=0)))
        tail_ref[n] = xz2[H - 8:, :]
        xcz_ref[0, :H] = s1
        xcz_ref[0, H:] = s2

        @pl.when(n == 0)
        def _():
            acc_ref[:H, :NB] = s1
            acc_ref[H:, :NB] = s2

        @pl.when(n == 1)
        def _():
            acc_ref[:H, NB:] = s1
            acc_ref[H:, NB:] = s2
            x_dbl = jnp.dot(acc_ref[:].astype(jnp.bfloat16),
                            xp_ref[...].astype(jnp.bfloat16),
                            preferred_element_type=jnp.float32)
            dlt_lin = jnp.dot(x_dbl[:, :DT_RANK_].astype(jnp.bfloat16),
                              dtw_ref[...].astype(jnp.bfloat16),
                              preferred_element_type=jnp.float32) + dtb_ref[...]
            dlt_ref[0] = _softplus(dlt_lin)
            bc_ref[0] = x_dbl[:, DT_RANK_:DT_RANK_ + 2 * N_STATE_]

    @pl.when(n >= 2)
    def _gate_half():
        xh = x_ref[0].astype(jnp.bfloat16)
        xz1 = jnp.dot(xh[:H], w_ref[...], preferred_element_type=jnp.float32)
        xz2 = jnp.dot(xh[H:], w_ref[...], preferred_element_type=jnp.float32)
        xcz_ref[0, :H] = _silu(xz1)
        xcz_ref[0, H:] = _silu(xz2)


def _k2(dlt_ref, xc_ref, zs_ref, bc_ref, at_ref, d_ref, out_ref, h_ref,
        hs_ref):
    l = pl.program_id(2)

    @pl.when(l == 0)
    def _():
        h_ref[:] = jnp.zeros_like(h_ref)

    a_mat = -jnp.exp(at_ref[:])
    d_vec = d_ref[:]
    lane_i = jax.lax.broadcasted_iota(jnp.int32, (8, 8 * N_STATE_), 1)
    row_i = jax.lax.broadcasted_iota(jnp.int32, (8, 8 * N_STATE_), 0)
    bdiag = (lane_i // N_STATE_) == row_i

    def group(i, _):
        base = pl.multiple_of(i * 8, 8)
        d8 = dlt_ref[0, pl.ds(base, 8), :]
        x8 = xc_ref[0, pl.ds(base, 8), :]
        z8 = zs_ref[0, pl.ds(base, 8), :]
        bc8 = bc_ref[0, pl.ds(base, 8), :]
        u8 = d8 * x8
        bdiag_b = jnp.where(bdiag, jnp.tile(bc8[:, :N_STATE_], (1, 8)), 0.0)
        cdiag = jnp.where(bdiag, jnp.tile(bc8[:, N_STATE_:], (1, 8)), 0.0)
        ub = jax.lax.dot_general(bdiag_b, u8, (((0,), (0,)), ((), ())),
                                 preferred_element_type=jnp.float32)
        h = h_ref[:]
        for j in range(8):
            dA = jnp.exp(d8[j:j + 1, :] * a_mat)
            h = dA * h + ub[j * N_STATE_:(j + 1) * N_STATE_, :]
            hs_ref[j * N_STATE_:(j + 1) * N_STATE_, :] = h
        h_ref[:] = h
        y8 = jnp.dot(cdiag, hs_ref[:], preferred_element_type=jnp.float32)
        out_ref[0, pl.ds(base, 8), :] = (y8 + x8 * d_vec) * z8
        return 0

    jax.lax.fori_loop(0, T2 // 8, group, 0)


def _k3(g_ref, w_ref, o_ref):
    o_ref[0] = jnp.dot(g_ref[0].astype(jnp.bfloat16), w_ref[...],
                       preferred_element_type=jnp.float32)


def kernel(x, in_proj_w, conv_w, conv_b, x_proj_w, dt_proj_w, dt_proj_b,
           A_log, D, out_proj_w):
    B, L, _ = x.shape
    in_w_bf = in_proj_w.astype(jnp.bfloat16)
    out_w_bf = out_proj_w.astype(jnp.bfloat16)
    cb2 = conv_b.reshape(1, D_INNER_)
    dtb2 = dt_proj_b.reshape(1, D_INNER_)
    a_t = jnp.transpose(A_log)
    d2 = D.reshape(1, D_INNER_)

    lt1 = L // T1
    xcz, dlt, bc = pl.pallas_call(
        _k1,
        grid=(B, lt1, 4),
        in_specs=[
            pl.BlockSpec((1, T1, D_MODEL_), lambda b, l, n: (b, l, 0)),
            pl.BlockSpec((D_MODEL_, NB), lambda b, l, n: (0, n)),
            pl.BlockSpec((NB, 4), lambda b, l, n: (n % 2, 0)),
            pl.BlockSpec((1, NB), lambda b, l, n: (0, n % 2)),
            pl.BlockSpec((D_INNER_, DT_RANK_ + 2 * N_STATE_),
                         lambda b, l, n: (0, 0)),
            pl.BlockSpec((DT_RANK_, D_INNER_), lambda b, l, n: (0, 0)),
            pl.BlockSpec((1, D_INNER_), lambda b, l, n: (0, 0)),
        ],
        out_specs=[
            pl.BlockSpec((1, T1, NB), lambda b, l, n: (b, l, n)),
            pl.BlockSpec((1, T1, D_INNER_), lambda b, l, n: (b, l, 0)),
            pl.BlockSpec((1, T1, 2 * N_STATE_), lambda b, l, n: (b, l, 0)),
        ],
        out_shape=[
            jax.ShapeDtypeStruct((B, L, 2 * D_INNER_), jnp.float32),
            jax.ShapeDtypeStruct((B, L, D_INNER_), jnp.float32),
            jax.ShapeDtypeStruct((B, L, 2 * N_STATE_), jnp.float32),
        ],
        scratch_shapes=[
            pltpu.VMEM((2, 8, NB), jnp.float32),
            pltpu.VMEM((T1, D_INNER_), jnp.float32),
        ],
        compiler_params=pltpu.CompilerParams(
            dimension_semantics=("parallel", "arbitrary", "arbitrary"),
            vmem_limit_bytes=52 * 1024 * 1024,
        ),
        name="ssm_pre",
    )(x, in_w_bf, conv_w, cb2, x_proj_w, dt_proj_w, dtb2)

    dt_blocks = D_INNER_ // DB
    lt2 = L // T2
    g = pl.pallas_call(
        _k2,
        grid=(B, dt_blocks, lt2),
        in_specs=[
            pl.BlockSpec((1, T2, DB), lambda b, d, l: (b, l, d)),
            pl.BlockSpec((1, T2, DB), lambda b, d, l: (b, l, d)),
            pl.BlockSpec((1, T2, DB), lambda b, d, l: (b, l, d + D_INNER_ // DB)),
            pl.BlockSpec((1, T2, 2 * N_STATE_), lambda b, d, l: (b, l, 0)),
            pl.BlockSpec((N_STATE_, DB), lambda b, d, l: (0, d)),
            pl.BlockSpec((1, DB), lambda b, d, l: (0, d)),
        ],
        out_specs=pl.BlockSpec((1, T2, DB), lambda b, d, l: (b, l, d)),
        out_shape=jax.ShapeDtypeStruct((B, L, D_INNER_), jnp.float32),
        scratch_shapes=[pltpu.VMEM((N_STATE_, DB), jnp.float32),
                        pltpu.VMEM((8 * N_STATE_, DB), jnp.float32)],
        compiler_params=pltpu.CompilerParams(
            dimension_semantics=("parallel", "parallel", "arbitrary"),
            vmem_limit_bytes=52 * 1024 * 1024,
        ),
        name="ssm_scan",
    )(dlt, xcz, xcz, bc, a_t, d2)

    lt3 = L // T3
    out = pl.pallas_call(
        _k3,
        grid=(B, lt3),
        in_specs=[
            pl.BlockSpec((1, T3, D_INNER_), lambda b, l: (b, l, 0)),
            pl.BlockSpec((D_INNER_, D_MODEL_), lambda b, l: (0, 0)),
        ],
        out_specs=pl.BlockSpec((1, T3, D_MODEL_), lambda b, l: (b, l, 0)),
        out_shape=jax.ShapeDtypeStruct((B, L, D_MODEL_), jnp.float32),
        compiler_params=pltpu.CompilerParams(
            dimension_semantics=("parallel", "arbitrary"),
            vmem_limit_bytes=52 * 1024 * 1024,
        ),
        name="ssm_out",
    )(g, out_w_bf)
    return out


# speedup vs baseline: 1.3411x; 1.3411x over previous
import jax
import jax.numpy as jnp
from jax.experimental import pallas as pl
from jax.experimental.pallas import tpu as pltpu

D_MODEL_ = 1024
D_INNER_ = 2048
N_STATE_ = 16
DT_RANK_ = 64
T1 = 512
NB = 1024
T2 = 256
DB = 2048
T3 = 512
G_ = 16


def _silu(v):
    return v * jax.nn.sigmoid(v)


def _softplus(v):
    return jnp.maximum(v, 0.0) + jnp.log1p(jnp.exp(-jnp.abs(v)))


def _k1(x_ref, w_ref, cw_ref, cb_ref, xp_ref, dtw_ref, dtb_ref,
        xcz_ref, dlt_ref, bc_ref, tail_ref, acc_ref):
    l = pl.program_id(1)
    n = pl.program_id(2)
    H = T1 // 2

    def _taps(ext):
        return (ext[0:H] * cw_ref[:, 0] + ext[1:H + 1] * cw_ref[:, 1]
                + ext[2:H + 2] * cw_ref[:, 2] + ext[3:H + 3] * cw_ref[:, 3]
                + cb_ref[...])

    @pl.when(n < 2)
    def _conv_half():
        xh = x_ref[0].astype(jnp.bfloat16)
        xz1 = jnp.dot(xh[:H], w_ref[...], preferred_element_type=jnp.float32)
        xz2 = jnp.dot(xh[H:], w_ref[...], preferred_element_type=jnp.float32)
        prev = tail_ref[n, 5:8, :]
        prev = jnp.where(l == 0, jnp.zeros_like(prev), prev)
        s1 = _silu(_taps(jnp.concatenate([prev, xz1], axis=0)))
        s2 = _silu(_taps(jnp.concatenate([xz1[H - 3:], xz2], axis=0)))
        tail_ref[n] = xz2[H - 8:, :]
        xcz_ref[0, :H] = s1
        xcz_ref[0, H:] = s2

        @pl.when(n == 0)
        def _():
            acc_ref[:H, :NB] = s1
            acc_ref[H:, :NB] = s2

        @pl.when(n == 1)
        def _():
            acc_ref[:H, NB:] = s1
            acc_ref[H:, NB:] = s2
            x_dbl = jnp.dot(acc_ref[:].astype(jnp.bfloat16),
                            xp_ref[...].astype(jnp.bfloat16),
                            preferred_element_type=jnp.float32)
            dlt_lin = jnp.dot(x_dbl[:, :DT_RANK_].astype(jnp.bfloat16),
                              dtw_ref[...].astype(jnp.bfloat16),
                              preferred_element_type=jnp.float32) + dtb_ref[...]
            dlt_ref[0] = _softplus(dlt_lin)
            bc_ref[0] = x_dbl[:, DT_RANK_:DT_RANK_ + 2 * N_STATE_]

    @pl.when(n >= 2)
    def _gate_half():
        xh = x_ref[0].astype(jnp.bfloat16)
        xz1 = jnp.dot(xh[:H], w_ref[...], preferred_element_type=jnp.float32)
        xz2 = jnp.dot(xh[H:], w_ref[...], preferred_element_type=jnp.float32)
        xcz_ref[0, :H] = _silu(xz1)
        xcz_ref[0, H:] = _silu(xz2)


def _k2(dlt_ref, xc_ref, zs_ref, bc_ref, at_ref, d_ref, out_ref, h_ref,
        hs_ref):
    l = pl.program_id(2)

    @pl.when(l == 0)
    def _():
        h_ref[:] = jnp.zeros_like(h_ref)

    a_mat = -jnp.exp(at_ref[:])
    d_vec = d_ref[:]
    lane_i = jax.lax.broadcasted_iota(jnp.int32, (G_, G_ * N_STATE_), 1)
    row_i = jax.lax.broadcasted_iota(jnp.int32, (G_, G_ * N_STATE_), 0)
    bdiag = (lane_i // N_STATE_) == row_i

    def group(i, _):
        base = pl.multiple_of(i * G_, 8)
        d8 = dlt_ref[0, pl.ds(base, G_), :]
        x8 = xc_ref[0, pl.ds(base, G_), :]
        z8 = zs_ref[0, pl.ds(base, G_), :]
        bc8 = bc_ref[0, pl.ds(base, G_), :]
        bcT = jnp.transpose(bc8)
        u8 = d8 * x8
        h = h_ref[:]
        for j in range(G_):
            dA = jnp.exp(d8[j:j + 1, :] * a_mat)
            h = dA * h + u8[j:j + 1, :] * bcT[0:N_STATE_, j:j + 1]
            hs_ref[j * N_STATE_:(j + 1) * N_STATE_, :] = h
        h_ref[:] = h
        cdiag = jnp.where(bdiag, jnp.tile(bc8[:, N_STATE_:], (1, G_)), 0.0)
        y8 = jnp.dot(cdiag, hs_ref[:], preferred_element_type=jnp.float32)
        out_ref[0, pl.ds(base, G_), :] = (y8 + x8 * d_vec) * z8
        return 0

    jax.lax.fori_loop(0, T2 // G_, group, 0)


def _k3(g_ref, w_ref, o_ref):
    o_ref[0] = jnp.dot(g_ref[0].astype(jnp.bfloat16), w_ref[...],
                       preferred_element_type=jnp.float32)


def kernel(x, in_proj_w, conv_w, conv_b, x_proj_w, dt_proj_w, dt_proj_b,
           A_log, D, out_proj_w):
    B, L, _ = x.shape
    in_w_bf = in_proj_w.astype(jnp.bfloat16)
    out_w_bf = out_proj_w.astype(jnp.bfloat16)
    cb2 = conv_b.reshape(1, D_INNER_)
    dtb2 = dt_proj_b.reshape(1, D_INNER_)
    a_t = jnp.transpose(A_log)
    d2 = D.reshape(1, D_INNER_)

    lt1 = L // T1
    xcz, dlt, bc = pl.pallas_call(
        _k1,
        grid=(B, lt1, 4),
        in_specs=[
            pl.BlockSpec((1, T1, D_MODEL_), lambda b, l, n: (b, l, 0)),
            pl.BlockSpec((D_MODEL_, NB), lambda b, l, n: (0, n)),
            pl.BlockSpec((NB, 4), lambda b, l, n: (n % 2, 0)),
            pl.BlockSpec((1, NB), lambda b, l, n: (0, n % 2)),
            pl.BlockSpec((D_INNER_, DT_RANK_ + 2 * N_STATE_),
                         lambda b, l, n: (0, 0)),
            pl.BlockSpec((DT_RANK_, D_INNER_), lambda b, l, n: (0, 0)),
            pl.BlockSpec((1, D_INNER_), lambda b, l, n: (0, 0)),
        ],
        out_specs=[
            pl.BlockSpec((1, T1, NB), lambda b, l, n: (b, l, n)),
            pl.BlockSpec((1, T1, D_INNER_), lambda b, l, n: (b, l, 0)),
            pl.BlockSpec((1, T1, 2 * N_STATE_), lambda b, l, n: (b, l, 0)),
        ],
        out_shape=[
            jax.ShapeDtypeStruct((B, L, 2 * D_INNER_), jnp.float32),
            jax.ShapeDtypeStruct((B, L, D_INNER_), jnp.float32),
            jax.ShapeDtypeStruct((B, L, 2 * N_STATE_), jnp.float32),
        ],
        scratch_shapes=[
            pltpu.VMEM((2, 8, NB), jnp.float32),
            pltpu.VMEM((T1, D_INNER_), jnp.float32),
        ],
        compiler_params=pltpu.CompilerParams(
            dimension_semantics=("parallel", "arbitrary", "arbitrary"),
            vmem_limit_bytes=52 * 1024 * 1024,
        ),
        name="ssm_pre",
    )(x, in_w_bf, conv_w, cb2, x_proj_w, dt_proj_w, dtb2)

    dt_blocks = D_INNER_ // DB
    lt2 = L // T2
    g = pl.pallas_call(
        _k2,
        grid=(B, dt_blocks, lt2),
        in_specs=[
            pl.BlockSpec((1, T2, DB), lambda b, d, l: (b, l, d)),
            pl.BlockSpec((1, T2, DB), lambda b, d, l: (b, l, d)),
            pl.BlockSpec((1, T2, DB), lambda b, d, l: (b, l, d + D_INNER_ // DB)),
            pl.BlockSpec((1, T2, 2 * N_STATE_), lambda b, d, l: (b, l, 0)),
            pl.BlockSpec((N_STATE_, DB), lambda b, d, l: (0, d)),
            pl.BlockSpec((1, DB), lambda b, d, l: (0, d)),
        ],
        out_specs=pl.BlockSpec((1, T2, DB), lambda b, d, l: (b, l, d)),
        out_shape=jax.ShapeDtypeStruct((B, L, D_INNER_), jnp.float32),
        scratch_shapes=[pltpu.VMEM((N_STATE_, DB), jnp.float32),
                        pltpu.VMEM((G_ * N_STATE_, DB), jnp.float32)],
        compiler_params=pltpu.CompilerParams(
            dimension_semantics=("parallel", "parallel", "arbitrary"),
            vmem_limit_bytes=52 * 1024 * 1024,
        ),
        name="ssm_scan",
    )(dlt, xcz, xcz, bc, a_t, d2)

    lt3 = L // T3
    out = pl.pallas_call(
        _k3,
        grid=(B, lt3),
        in_specs=[
            pl.BlockSpec((1, T3, D_INNER_), lambda b, l: (b, l, 0)),
            pl.BlockSpec((D_INNER_, D_MODEL_), lambda b, l: (0, 0)),
        ],
        out_specs=pl.BlockSpec((1, T3, D_MODEL_), lambda b, l: (b, l, 0)),
        out_shape=jax.ShapeDtypeStruct((B, L, D_MODEL_), jnp.float32),
        compiler_params=pltpu.CompilerParams(
            dimension_semantics=("parallel", "arbitrary"),
            vmem_limit_bytes=52 * 1024 * 1024,
        ),
        name="ssm_out",
    )(g, out_w_bf)
    return out


# speedup vs baseline: 1.4266x; 1.0638x over previous
import jax
import jax.numpy as jnp
from jax.experimental import pallas as pl
from jax.experimental.pallas import tpu as pltpu

D_MODEL_ = 1024
D_INNER_ = 2048
N_STATE_ = 16
DT_RANK_ = 64
T1 = 512
NB = 1024
T2 = 256
DB = 2048
T3 = 512
G_ = 32


def _silu(v):
    return v * jax.nn.sigmoid(v)


def _softplus(v):
    return jnp.maximum(v, 0.0) + jnp.log1p(jnp.exp(-jnp.abs(v)))


def _k1(x_ref, w_ref, cw_ref, cb_ref, xp_ref, dtw_ref, dtb_ref,
        xcz_ref, dlt_ref, bc_ref, tail_ref, acc_ref):
    l = pl.program_id(1)
    n = pl.program_id(2)
    H = T1 // 2

    def _taps(ext):
        return (ext[0:H] * cw_ref[:, 0] + ext[1:H + 1] * cw_ref[:, 1]
                + ext[2:H + 2] * cw_ref[:, 2] + ext[3:H + 3] * cw_ref[:, 3]
                + cb_ref[...])

    @pl.when(n < 2)
    def _conv_half():
        xh = x_ref[0].astype(jnp.bfloat16)
        xz1 = jnp.dot(xh[:H], w_ref[...], preferred_element_type=jnp.float32)
        xz2 = jnp.dot(xh[H:], w_ref[...], preferred_element_type=jnp.float32)
        prev = tail_ref[n, 5:8, :]
        prev = jnp.where(l == 0, jnp.zeros_like(prev), prev)
        s1 = _silu(_taps(jnp.concatenate([prev, xz1], axis=0)))
        s2 = _silu(_taps(jnp.concatenate([xz1[H - 3:], xz2], axis=0)))
        tail_ref[n] = xz2[H - 8:, :]
        xcz_ref[0, :H] = s1
        xcz_ref[0, H:] = s2

        @pl.when(n == 0)
        def _():
            acc_ref[:H, :NB] = s1
            acc_ref[H:, :NB] = s2

        @pl.when(n == 1)
        def _():
            acc_ref[:H, NB:] = s1
            acc_ref[H:, NB:] = s2
            x_dbl = jnp.dot(acc_ref[:].astype(jnp.bfloat16),
                            xp_ref[...].astype(jnp.bfloat16),
                            preferred_element_type=jnp.float32)
            dlt_lin = jnp.dot(x_dbl[:, :DT_RANK_].astype(jnp.bfloat16),
                              dtw_ref[...].astype(jnp.bfloat16),
                              preferred_element_type=jnp.float32) + dtb_ref[...]
            dlt_ref[0] = _softplus(dlt_lin)
            bc_ref[0] = x_dbl[:, DT_RANK_:DT_RANK_ + 2 * N_STATE_]

    @pl.when(n >= 2)
    def _gate_half():
        xh = x_ref[0].astype(jnp.bfloat16)
        xz1 = jnp.dot(xh[:H], w_ref[...], preferred_element_type=jnp.float32)
        xz2 = jnp.dot(xh[H:], w_ref[...], preferred_element_type=jnp.float32)
        xcz_ref[0, :H] = _silu(xz1)
        xcz_ref[0, H:] = _silu(xz2)


def _k2(dlt_ref, xc_ref, zs_ref, bc_ref, at_ref, d_ref, out_ref, h_ref,
        hs_ref):
    l = pl.program_id(2)

    @pl.when(l == 0)
    def _():
        h_ref[:] = jnp.zeros_like(h_ref)

    a_mat = -jnp.exp(at_ref[:])
    d_vec = d_ref[:]
    lane_i = jax.lax.broadcasted_iota(jnp.int32, (G_, G_ * N_STATE_), 1)
    row_i = jax.lax.broadcasted_iota(jnp.int32, (G_, G_ * N_STATE_), 0)
    bdiag = (lane_i // N_STATE_) == row_i

    def group(i, _):
        base = pl.multiple_of(i * G_, 8)
        d8 = dlt_ref[0, pl.ds(base, G_), :]
        x8 = xc_ref[0, pl.ds(base, G_), :]
        z8 = zs_ref[0, pl.ds(base, G_), :]
        bc8 = bc_ref[0, pl.ds(base, G_), :]
        bcT = jnp.transpose(bc8)
        u8 = d8 * x8
        h = h_ref[:]
        for j in range(G_):
            dA = jnp.exp(d8[j:j + 1, :] * a_mat)
            h = dA * h + u8[j:j + 1, :] * bcT[0:N_STATE_, j:j + 1]
            hs_ref[j * N_STATE_:(j + 1) * N_STATE_, :] = h
        h_ref[:] = h
        cdiag = jnp.where(bdiag, jnp.tile(bc8[:, N_STATE_:], (1, G_)), 0.0)
        y8 = jnp.dot(cdiag, hs_ref[:], preferred_element_type=jnp.float32)
        out_ref[0, pl.ds(base, G_), :] = (y8 + x8 * d_vec) * z8
        return 0

    jax.lax.fori_loop(0, T2 // G_, group, 0)


def _k3(g_ref, w_ref, o_ref):
    o_ref[0] = jnp.dot(g_ref[0].astype(jnp.bfloat16), w_ref[...],
                       preferred_element_type=jnp.float32)


def kernel(x, in_proj_w, conv_w, conv_b, x_proj_w, dt_proj_w, dt_proj_b,
           A_log, D, out_proj_w):
    B, L, _ = x.shape
    in_w_bf = in_proj_w.astype(jnp.bfloat16)
    out_w_bf = out_proj_w.astype(jnp.bfloat16)
    cb2 = conv_b.reshape(1, D_INNER_)
    dtb2 = dt_proj_b.reshape(1, D_INNER_)
    a_t = jnp.transpose(A_log)
    d2 = D.reshape(1, D_INNER_)

    lt1 = L // T1
    xcz, dlt, bc = pl.pallas_call(
        _k1,
        grid=(B, lt1, 4),
        in_specs=[
            pl.BlockSpec((1, T1, D_MODEL_), lambda b, l, n: (b, l, 0)),
            pl.BlockSpec((D_MODEL_, NB), lambda b, l, n: (0, n)),
            pl.BlockSpec((NB, 4), lambda b, l, n: (n % 2, 0)),
            pl.BlockSpec((1, NB), lambda b, l, n: (0, n % 2)),
            pl.BlockSpec((D_INNER_, DT_RANK_ + 2 * N_STATE_),
                         lambda b, l, n: (0, 0)),
            pl.BlockSpec((DT_RANK_, D_INNER_), lambda b, l, n: (0, 0)),
            pl.BlockSpec((1, D_INNER_), lambda b, l, n: (0, 0)),
        ],
        out_specs=[
            pl.BlockSpec((1, T1, NB), lambda b, l, n: (b, l, n)),
            pl.BlockSpec((1, T1, D_INNER_), lambda b, l, n: (b, l, 0)),
            pl.BlockSpec((1, T1, 2 * N_STATE_), lambda b, l, n: (b, l, 0)),
        ],
        out_shape=[
            jax.ShapeDtypeStruct((B, L, 2 * D_INNER_), jnp.float32),
            jax.ShapeDtypeStruct((B, L, D_INNER_), jnp.float32),
            jax.ShapeDtypeStruct((B, L, 2 * N_STATE_), jnp.float32),
        ],
        scratch_shapes=[
            pltpu.VMEM((2, 8, NB), jnp.float32),
            pltpu.VMEM((T1, D_INNER_), jnp.float32),
        ],
        compiler_params=pltpu.CompilerParams(
            dimension_semantics=("parallel", "arbitrary", "arbitrary"),
            vmem_limit_bytes=52 * 1024 * 1024,
        ),
        name="ssm_pre",
    )(x, in_w_bf, conv_w, cb2, x_proj_w, dt_proj_w, dtb2)

    dt_blocks = D_INNER_ // DB
    lt2 = L // T2
    g = pl.pallas_call(
        _k2,
        grid=(B, dt_blocks, lt2),
        in_specs=[
            pl.BlockSpec((1, T2, DB), lambda b, d, l: (b, l, d)),
            pl.BlockSpec((1, T2, DB), lambda b, d, l: (b, l, d)),
            pl.BlockSpec((1, T2, DB), lambda b, d, l: (b, l, d + D_INNER_ // DB)),
            pl.BlockSpec((1, T2, 2 * N_STATE_), lambda b, d, l: (b, l, 0)),
            pl.BlockSpec((N_STATE_, DB), lambda b, d, l: (0, d)),
            pl.BlockSpec((1, DB), lambda b, d, l: (0, d)),
        ],
        out_specs=pl.BlockSpec((1, T2, DB), lambda b, d, l: (b, l, d)),
        out_shape=jax.ShapeDtypeStruct((B, L, D_INNER_), jnp.float32),
        scratch_shapes=[pltpu.VMEM((N_STATE_, DB), jnp.float32),
                        pltpu.VMEM((G_ * N_STATE_, DB), jnp.float32)],
        compiler_params=pltpu.CompilerParams(
            dimension_semantics=("parallel", "parallel", "arbitrary"),
            vmem_limit_bytes=52 * 1024 * 1024,
        ),
        name="ssm_scan",
    )(dlt, xcz, xcz, bc, a_t, d2)

    lt3 = L // T3
    out = pl.pallas_call(
        _k3,
        grid=(B, lt3),
        in_specs=[
            pl.BlockSpec((1, T3, D_INNER_), lambda b, l: (b, l, 0)),
            pl.BlockSpec((D_INNER_, D_MODEL_), lambda b, l: (0, 0)),
        ],
        out_specs=pl.BlockSpec((1, T3, D_MODEL_), lambda b, l: (b, l, 0)),
        out_shape=jax.ShapeDtypeStruct((B, L, D_MODEL_), jnp.float32),
        compiler_params=pltpu.CompilerParams(
            dimension_semantics=("parallel", "arbitrary"),
            vmem_limit_bytes=52 * 1024 * 1024,
        ),
        name="ssm_out",
    )(g, out_w_bf)
    return out


# speedup vs baseline: 1.4652x; 1.0270x over previous
import jax
import jax.numpy as jnp
from jax.experimental import pallas as pl
from jax.experimental.pallas import tpu as pltpu

D_MODEL_ = 1024
D_INNER_ = 2048
N_STATE_ = 16
DT_RANK_ = 64
T1 = 512
NB = 1024
T2 = 256
DB = 2048
T3 = 512
G_ = 64


def _silu(v):
    return v * jax.nn.sigmoid(v)


def _softplus(v):
    return jnp.maximum(v, 0.0) + jnp.log1p(jnp.exp(-jnp.abs(v)))


def _k1(x_ref, w_ref, cw_ref, cb_ref, xp_ref, dtw_ref, dtb_ref,
        xcz_ref, dlt_ref, bc_ref, tail_ref, acc_ref):
    l = pl.program_id(1)
    n = pl.program_id(2)
    H = T1 // 2

    def _taps(ext):
        return (ext[0:H] * cw_ref[:, 0] + ext[1:H + 1] * cw_ref[:, 1]
                + ext[2:H + 2] * cw_ref[:, 2] + ext[3:H + 3] * cw_ref[:, 3]
                + cb_ref[...])

    @pl.when(n < 2)
    def _conv_half():
        xh = x_ref[0].astype(jnp.bfloat16)
        xz1 = jnp.dot(xh[:H], w_ref[...], preferred_element_type=jnp.float32)
        xz2 = jnp.dot(xh[H:], w_ref[...], preferred_element_type=jnp.float32)
        prev = tail_ref[n, 5:8, :]
        prev = jnp.where(l == 0, jnp.zeros_like(prev), prev)
        s1 = _silu(_taps(jnp.concatenate([prev, xz1], axis=0)))
        s2 = _silu(_taps(jnp.concatenate([xz1[H - 3:], xz2], axis=0)))
        tail_ref[n] = xz2[H - 8:, :]
        xcz_ref[0, :H] = s1
        xcz_ref[0, H:] = s2

        @pl.when(n == 0)
        def _():
            acc_ref[:H, :NB] = s1
            acc_ref[H:, :NB] = s2

        @pl.when(n == 1)
        def _():
            acc_ref[:H, NB:] = s1
            acc_ref[H:, NB:] = s2
            x_dbl = jnp.dot(acc_ref[:].astype(jnp.bfloat16),
                            xp_ref[...].astype(jnp.bfloat16),
                            preferred_element_type=jnp.float32)
            dlt_lin = jnp.dot(x_dbl[:, :DT_RANK_].astype(jnp.bfloat16),
                              dtw_ref[...].astype(jnp.bfloat16),
                              preferred_element_type=jnp.float32) + dtb_ref[...]
            dlt_ref[0] = _softplus(dlt_lin)
            bc_ref[0] = x_dbl[:, DT_RANK_:DT_RANK_ + 2 * N_STATE_]

    @pl.when(n >= 2)
    def _gate_half():
        xh = x_ref[0].astype(jnp.bfloat16)
        xz1 = jnp.dot(xh[:H], w_ref[...], preferred_element_type=jnp.float32)
        xz2 = jnp.dot(xh[H:], w_ref[...], preferred_element_type=jnp.float32)
        xcz_ref[0, :H] = _silu(xz1)
        xcz_ref[0, H:] = _silu(xz2)


def _k2(dlt_ref, xc_ref, zs_ref, bc_ref, at_ref, d_ref, out_ref, h_ref,
        hs_ref):
    l = pl.program_id(2)

    @pl.when(l == 0)
    def _():
        h_ref[:] = jnp.zeros_like(h_ref)

    a_mat = -jnp.exp(at_ref[:])
    d_vec = d_ref[:]
    lane_i = jax.lax.broadcasted_iota(jnp.int32, (G_, G_ * N_STATE_), 1)
    row_i = jax.lax.broadcasted_iota(jnp.int32, (G_, G_ * N_STATE_), 0)
    bdiag = (lane_i // N_STATE_) == row_i

    def group(i, _):
        base = pl.multiple_of(i * G_, 8)
        d8 = dlt_ref[0, pl.ds(base, G_), :]
        x8 = xc_ref[0, pl.ds(base, G_), :]
        z8 = zs_ref[0, pl.ds(base, G_), :]
        bc8 = bc_ref[0, pl.ds(base, G_), :]
        bcT = jnp.transpose(bc8)
        u8 = d8 * x8
        h = h_ref[:]
        for j in range(G_):
            dA = jnp.exp(d8[j:j + 1, :] * a_mat)
            h = dA * h + u8[j:j + 1, :] * bcT[0:N_STATE_, j:j + 1]
            hs_ref[j * N_STATE_:(j + 1) * N_STATE_, :] = h
        h_ref[:] = h
        cdiag = jnp.where(bdiag, jnp.tile(bc8[:, N_STATE_:], (1, G_)), 0.0)
        y8 = jnp.dot(cdiag, hs_ref[:], preferred_element_type=jnp.float32)
        out_ref[0, pl.ds(base, G_), :] = (y8 + x8 * d_vec) * z8
        return 0

    jax.lax.fori_loop(0, T2 // G_, group, 0)


def _k3(g_ref, w_ref, o_ref):
    o_ref[0] = jnp.dot(g_ref[0].astype(jnp.bfloat16), w_ref[...],
                       preferred_element_type=jnp.float32)


def kernel(x, in_proj_w, conv_w, conv_b, x_proj_w, dt_proj_w, dt_proj_b,
           A_log, D, out_proj_w):
    B, L, _ = x.shape
    in_w_bf = in_proj_w.astype(jnp.bfloat16)
    out_w_bf = out_proj_w.astype(jnp.bfloat16)
    cb2 = conv_b.reshape(1, D_INNER_)
    dtb2 = dt_proj_b.reshape(1, D_INNER_)
    a_t = jnp.transpose(A_log)
    d2 = D.reshape(1, D_INNER_)

    lt1 = L // T1
    xcz, dlt, bc = pl.pallas_call(
        _k1,
        grid=(B, lt1, 4),
        in_specs=[
            pl.BlockSpec((1, T1, D_MODEL_), lambda b, l, n: (b, l, 0)),
            pl.BlockSpec((D_MODEL_, NB), lambda b, l, n: (0, n)),
            pl.BlockSpec((NB, 4), lambda b, l, n: (n % 2, 0)),
            pl.BlockSpec((1, NB), lambda b, l, n: (0, n % 2)),
            pl.BlockSpec((D_INNER_, DT_RANK_ + 2 * N_STATE_),
                         lambda b, l, n: (0, 0)),
            pl.BlockSpec((DT_RANK_, D_INNER_), lambda b, l, n: (0, 0)),
            pl.BlockSpec((1, D_INNER_), lambda b, l, n: (0, 0)),
        ],
        out_specs=[
            pl.BlockSpec((1, T1, NB), lambda b, l, n: (b, l, n)),
            pl.BlockSpec((1, T1, D_INNER_), lambda b, l, n: (b, l, 0)),
            pl.BlockSpec((1, T1, 2 * N_STATE_), lambda b, l, n: (b, l, 0)),
        ],
        out_shape=[
            jax.ShapeDtypeStruct((B, L, 2 * D_INNER_), jnp.float32),
            jax.ShapeDtypeStruct((B, L, D_INNER_), jnp.float32),
            jax.ShapeDtypeStruct((B, L, 2 * N_STATE_), jnp.float32),
        ],
        scratch_shapes=[
            pltpu.VMEM((2, 8, NB), jnp.float32),
            pltpu.VMEM((T1, D_INNER_), jnp.float32),
        ],
        compiler_params=pltpu.CompilerParams(
            dimension_semantics=("parallel", "arbitrary", "arbitrary"),
            vmem_limit_bytes=52 * 1024 * 1024,
        ),
        name="ssm_pre",
    )(x, in_w_bf, conv_w, cb2, x_proj_w, dt_proj_w, dtb2)

    dt_blocks = D_INNER_ // DB
    lt2 = L // T2
    g = pl.pallas_call(
        _k2,
        grid=(B, dt_blocks, lt2),
        in_specs=[
            pl.BlockSpec((1, T2, DB), lambda b, d, l: (b, l, d)),
            pl.BlockSpec((1, T2, DB), lambda b, d, l: (b, l, d)),
            pl.BlockSpec((1, T2, DB), lambda b, d, l: (b, l, d + D_INNER_ // DB)),
            pl.BlockSpec((1, T2, 2 * N_STATE_), lambda b, d, l: (b, l, 0)),
            pl.BlockSpec((N_STATE_, DB), lambda b, d, l: (0, d)),
            pl.BlockSpec((1, DB), lambda b, d, l: (0, d)),
        ],
        out_specs=pl.BlockSpec((1, T2, DB), lambda b, d, l: (b, l, d)),
        out_shape=jax.ShapeDtypeStruct((B, L, D_INNER_), jnp.float32),
        scratch_shapes=[pltpu.VMEM((N_STATE_, DB), jnp.float32),
                        pltpu.VMEM((G_ * N_STATE_, DB), jnp.float32)],
        compiler_params=pltpu.CompilerParams(
            dimension_semantics=("parallel", "parallel", "arbitrary"),
            vmem_limit_bytes=52 * 1024 * 1024,
        ),
        name="ssm_scan",
    )(dlt, xcz, xcz, bc, a_t, d2)

    lt3 = L // T3
    out = pl.pallas_call(
        _k3,
        grid=(B, lt3),
        in_specs=[
            pl.BlockSpec((1, T3, D_INNER_), lambda b, l: (b, l, 0)),
            pl.BlockSpec((D_INNER_, D_MODEL_), lambda b, l: (0, 0)),
        ],
        out_specs=pl.BlockSpec((1, T3, D_MODEL_), lambda b, l: (b, l, 0)),
        out_shape=jax.ShapeDtypeStruct((B, L, D_MODEL_), jnp.float32),
        compiler_params=pltpu.CompilerParams(
            dimension_semantics=("parallel", "arbitrary"),
            vmem_limit_bytes=52 * 1024 * 1024,
        ),
        name="ssm_out",
    )(g, out_w_bf)
    return out


# speedup vs baseline: 1.5276x; 1.0426x over previous
import jax
import jax.numpy as jnp
from jax.experimental import pallas as pl
from jax.experimental.pallas import tpu as pltpu

D_MODEL_ = 1024
D_INNER_ = 2048
N_STATE_ = 16
DT_RANK_ = 64
T1 = 512
NB = 1024
T2 = 512
DB = 2048
T3 = 512
G_ = 64


def _silu(v):
    return v * jax.nn.sigmoid(v)


def _softplus(v):
    return jnp.maximum(v, 0.0) + jnp.log1p(jnp.exp(-jnp.abs(v)))


def _k1(x_ref, w_ref, cw_ref, cb_ref, xp_ref, dtw_ref, dtb_ref,
        xcz_ref, dlt_ref, bc_ref, tail_ref, acc_ref):
    l = pl.program_id(1)
    n = pl.program_id(2)
    H = T1 // 2

    def _taps(ext):
        return (ext[0:H] * cw_ref[:, 0] + ext[1:H + 1] * cw_ref[:, 1]
                + ext[2:H + 2] * cw_ref[:, 2] + ext[3:H + 3] * cw_ref[:, 3]
                + cb_ref[...])

    @pl.when(n < 2)
    def _conv_half():
        xh = x_ref[0].astype(jnp.bfloat16)
        xz1 = jnp.dot(xh[:H], w_ref[...], preferred_element_type=jnp.float32)
        xz2 = jnp.dot(xh[H:], w_ref[...], preferred_element_type=jnp.float32)
        prev = tail_ref[n, 5:8, :]
        prev = jnp.where(l == 0, jnp.zeros_like(prev), prev)
        s1 = _silu(_taps(jnp.concatenate([prev, xz1], axis=0)))
        s2 = _silu(_taps(jnp.concatenate([xz1[H - 3:], xz2], axis=0)))
        tail_ref[n] = xz2[H - 8:, :]
        xcz_ref[0, :H] = s1
        xcz_ref[0, H:] = s2

        @pl.when(n == 0)
        def _():
            acc_ref[:H, :NB] = s1
            acc_ref[H:, :NB] = s2

        @pl.when(n == 1)
        def _():
            acc_ref[:H, NB:] = s1
            acc_ref[H:, NB:] = s2
            x_dbl = jnp.dot(acc_ref[:].astype(jnp.bfloat16),
                            xp_ref[...].astype(jnp.bfloat16),
                            preferred_element_type=jnp.float32)
            dlt_lin = jnp.dot(x_dbl[:, :DT_RANK_].astype(jnp.bfloat16),
                              dtw_ref[...].astype(jnp.bfloat16),
                              preferred_element_type=jnp.float32) + dtb_ref[...]
            dlt_ref[0] = _softplus(dlt_lin)
            bc_ref[0] = x_dbl[:, DT_RANK_:DT_RANK_ + 2 * N_STATE_]

    @pl.when(n >= 2)
    def _gate_half():
        xh = x_ref[0].astype(jnp.bfloat16)
        xz1 = jnp.dot(xh[:H], w_ref[...], preferred_element_type=jnp.float32)
        xz2 = jnp.dot(xh[H:], w_ref[...], preferred_element_type=jnp.float32)
        xcz_ref[0, :H] = _silu(xz1)
        xcz_ref[0, H:] = _silu(xz2)


def _k2(dlt_ref, xc_ref, zs_ref, bc_ref, at_ref, d_ref, out_ref, h_ref,
        hs_ref):
    l = pl.program_id(2)

    @pl.when(l == 0)
    def _():
        h_ref[:] = jnp.zeros_like(h_ref)

    a_mat = jnp.exp(at_ref[:]) * (-1.4426950408889634)
    d_vec = d_ref[:]
    lane_i = jax.lax.broadcasted_iota(jnp.int32, (G_, G_ * N_STATE_), 1)
    row_i = jax.lax.broadcasted_iota(jnp.int32, (G_, G_ * N_STATE_), 0)
    bdiag = (lane_i // N_STATE_) == row_i

    def group(i, _):
        base = pl.multiple_of(i * G_, 8)
        d8 = dlt_ref[0, pl.ds(base, G_), :]
        x8 = xc_ref[0, pl.ds(base, G_), :]
        z8 = zs_ref[0, pl.ds(base, G_), :]
        bc8 = bc_ref[0, pl.ds(base, G_), :]
        bcT = jnp.transpose(bc8)
        u8 = d8 * x8
        h = h_ref[:]
        for j in range(G_):
            dA = jnp.exp2(d8[j:j + 1, :] * a_mat)
            h = dA * h + u8[j:j + 1, :] * bcT[0:N_STATE_, j:j + 1]
            hs_ref[j * N_STATE_:(j + 1) * N_STATE_, :] = h
        h_ref[:] = h
        cdiag = jnp.where(bdiag, jnp.tile(bc8[:, N_STATE_:], (1, G_)), 0.0)
        y8 = jnp.dot(cdiag, hs_ref[:], preferred_element_type=jnp.float32)
        out_ref[0, pl.ds(base, G_), :] = (y8 + x8 * d_vec) * z8
        return 0

    jax.lax.fori_loop(0, T2 // G_, group, 0)


def _k3(g_ref, w_ref, o_ref):
    o_ref[0] = jnp.dot(g_ref[0].astype(jnp.bfloat16), w_ref[...],
                       preferred_element_type=jnp.float32)


def kernel(x, in_proj_w, conv_w, conv_b, x_proj_w, dt_proj_w, dt_proj_b,
           A_log, D, out_proj_w):
    B, L, _ = x.shape
    in_w_bf = in_proj_w.astype(jnp.bfloat16)
    out_w_bf = out_proj_w.astype(jnp.bfloat16)
    cb2 = conv_b.reshape(1, D_INNER_)
    dtb2 = dt_proj_b.reshape(1, D_INNER_)
    a_t = jnp.transpose(A_log)
    d2 = D.reshape(1, D_INNER_)

    lt1 = L // T1
    xcz, dlt, bc = pl.pallas_call(
        _k1,
        grid=(B, lt1, 4),
        in_specs=[
            pl.BlockSpec((1, T1, D_MODEL_), lambda b, l, n: (b, l, 0)),
            pl.BlockSpec((D_MODEL_, NB), lambda b, l, n: (0, n)),
            pl.BlockSpec((NB, 4), lambda b, l, n: (n % 2, 0)),
            pl.BlockSpec((1, NB), lambda b, l, n: (0, n % 2)),
            pl.BlockSpec((D_INNER_, DT_RANK_ + 2 * N_STATE_),
                         lambda b, l, n: (0, 0)),
            pl.BlockSpec((DT_RANK_, D_INNER_), lambda b, l, n: (0, 0)),
            pl.BlockSpec((1, D_INNER_), lambda b, l, n: (0, 0)),
        ],
        out_specs=[
            pl.BlockSpec((1, T1, NB), lambda b, l, n: (b, l, n)),
            pl.BlockSpec((1, T1, D_INNER_), lambda b, l, n: (b, l, 0)),
            pl.BlockSpec((1, T1, 2 * N_STATE_), lambda b, l, n: (b, l, 0)),
        ],
        out_shape=[
            jax.ShapeDtypeStruct((B, L, 2 * D_INNER_), jnp.float32),
            jax.ShapeDtypeStruct((B, L, D_INNER_), jnp.float32),
            jax.ShapeDtypeStruct((B, L, 2 * N_STATE_), jnp.float32),
        ],
        scratch_shapes=[
            pltpu.VMEM((2, 8, NB), jnp.float32),
            pltpu.VMEM((T1, D_INNER_), jnp.float32),
        ],
        compiler_params=pltpu.CompilerParams(
            dimension_semantics=("parallel", "arbitrary", "arbitrary"),
            vmem_limit_bytes=52 * 1024 * 1024,
        ),
        name="ssm_pre",
    )(x, in_w_bf, conv_w, cb2, x_proj_w, dt_proj_w, dtb2)

    dt_blocks = D_INNER_ // DB
    lt2 = L // T2
    g = pl.pallas_call(
        _k2,
        grid=(B, dt_blocks, lt2),
        in_specs=[
            pl.BlockSpec((1, T2, DB), lambda b, d, l: (b, l, d)),
            pl.BlockSpec((1, T2, DB), lambda b, d, l: (b, l, d)),
            pl.BlockSpec((1, T2, DB), lambda b, d, l: (b, l, d + D_INNER_ // DB)),
            pl.BlockSpec((1, T2, 2 * N_STATE_), lambda b, d, l: (b, l, 0)),
            pl.BlockSpec((N_STATE_, DB), lambda b, d, l: (0, d)),
            pl.BlockSpec((1, DB), lambda b, d, l: (0, d)),
        ],
        out_specs=pl.BlockSpec((1, T2, DB), lambda b, d, l: (b, l, d)),
        out_shape=jax.ShapeDtypeStruct((B, L, D_INNER_), jnp.float32),
        scratch_shapes=[pltpu.VMEM((N_STATE_, DB), jnp.float32),
                        pltpu.VMEM((G_ * N_STATE_, DB), jnp.float32)],
        compiler_params=pltpu.CompilerParams(
            dimension_semantics=("parallel", "parallel", "arbitrary"),
            vmem_limit_bytes=52 * 1024 * 1024,
        ),
        name="ssm_scan",
    )(dlt, xcz, xcz, bc, a_t, d2)

    lt3 = L // T3
    out = pl.pallas_call(
        _k3,
        grid=(B, lt3),
        in_specs=[
            pl.BlockSpec((1, T3, D_INNER_), lambda b, l: (b, l, 0)),
            pl.BlockSpec((D_INNER_, D_MODEL_), lambda b, l: (0, 0)),
        ],
        out_specs=pl.BlockSpec((1, T3, D_MODEL_), lambda b, l: (b, l, 0)),
        out_shape=jax.ShapeDtypeStruct((B, L, D_MODEL_), jnp.float32),
        compiler_params=pltpu.CompilerParams(
            dimension_semantics=("parallel", "arbitrary"),
            vmem_limit_bytes=52 * 1024 * 1024,
        ),
        name="ssm_out",
    )(g, out_w_bf)
    return out


# speedup vs baseline: 1.5544x; 1.0175x over previous
import jax
import jax.numpy as jnp
from jax.experimental import pallas as pl
from jax.experimental.pallas import tpu as pltpu

D_MODEL_ = 1024
D_INNER_ = 2048
N_STATE_ = 16
DT_RANK_ = 64
T1 = 512
NB = 1024
T2 = 512
DB = 2048
T3 = 512
G_ = 64


_LOG2E = 1.4426950408889634
_LN2 = 0.6931471805599453


def _silu(v):
    return v / (1.0 + jnp.exp2(v * (-_LOG2E)))


def _softplus(v):
    t = jnp.exp2(jnp.abs(v) * (-_LOG2E))
    return jnp.maximum(v, 0.0) + jnp.log2(1.0 + t) * _LN2


def _k1(x_ref, w_ref, cw_ref, cb_ref, xp_ref, dtw_ref, dtb_ref,
        xcz_ref, dlt_ref, bc_ref, tail_ref, acc_ref):
    l = pl.program_id(1)
    n = pl.program_id(2)
    H = T1 // 2

    def _taps(ext):
        return (ext[0:H] * cw_ref[:, 0] + ext[1:H + 1] * cw_ref[:, 1]
                + ext[2:H + 2] * cw_ref[:, 2] + ext[3:H + 3] * cw_ref[:, 3]
                + cb_ref[...])

    @pl.when(n < 2)
    def _conv_half():
        xh = x_ref[0].astype(jnp.bfloat16)
        xz1 = jnp.dot(xh[:H], w_ref[...], preferred_element_type=jnp.float32)
        xz2 = jnp.dot(xh[H:], w_ref[...], preferred_element_type=jnp.float32)
        prev = tail_ref[n, 5:8, :]
        prev = jnp.where(l == 0, jnp.zeros_like(prev), prev)
        s1 = _silu(_taps(jnp.concatenate([prev, xz1], axis=0)))
        s2 = _silu(_taps(jnp.concatenate([xz1[H - 3:], xz2], axis=0)))
        tail_ref[n] = xz2[H - 8:, :]
        xcz_ref[0, :H] = s1
        xcz_ref[0, H:] = s2

        @pl.when(n == 0)
        def _():
            acc_ref[:H, :NB] = s1
            acc_ref[H:, :NB] = s2

        @pl.when(n == 1)
        def _():
            acc_ref[:H, NB:] = s1
            acc_ref[H:, NB:] = s2
            x_dbl = jnp.dot(acc_ref[:].astype(jnp.bfloat16),
                            xp_ref[...].astype(jnp.bfloat16),
                            preferred_element_type=jnp.float32)
            dlt_lin = jnp.dot(x_dbl[:, :DT_RANK_].astype(jnp.bfloat16),
                              dtw_ref[...].astype(jnp.bfloat16),
                              preferred_element_type=jnp.float32) + dtb_ref[...]
            dlt_ref[0] = _softplus(dlt_lin)
            bc_ref[0] = x_dbl[:, DT_RANK_:DT_RANK_ + 2 * N_STATE_]

    @pl.when(n >= 2)
    def _gate_half():
        xh = x_ref[0].astype(jnp.bfloat16)
        xz1 = jnp.dot(xh[:H], w_ref[...], preferred_element_type=jnp.float32)
        xz2 = jnp.dot(xh[H:], w_ref[...], preferred_element_type=jnp.float32)
        xcz_ref[0, :H] = _silu(xz1)
        xcz_ref[0, H:] = _silu(xz2)


def _k2(dlt_ref, xc_ref, zs_ref, bc_ref, at_ref, d_ref, out_ref, h_ref,
        hs_ref):
    l = pl.program_id(2)

    @pl.when(l == 0)
    def _():
        h_ref[:] = jnp.zeros_like(h_ref)

    a_mat = jnp.exp(at_ref[:]) * (-1.4426950408889634)
    d_vec = d_ref[:]
    lane_i = jax.lax.broadcasted_iota(jnp.int32, (G_, G_ * N_STATE_), 1)
    row_i = jax.lax.broadcasted_iota(jnp.int32, (G_, G_ * N_STATE_), 0)
    bdiag = (lane_i // N_STATE_) == row_i

    def group(i, _):
        base = pl.multiple_of(i * G_, 8)
        d8 = dlt_ref[0, pl.ds(base, G_), :]
        x8 = xc_ref[0, pl.ds(base, G_), :]
        z8 = zs_ref[0, pl.ds(base, G_), :]
        bc8 = bc_ref[0, pl.ds(base, G_), :]
        bcT = jnp.transpose(bc8)
        u8 = d8 * x8
        h = h_ref[:]
        for j in range(G_):
            dA = jnp.exp2(d8[j:j + 1, :] * a_mat)
            h = dA * h + u8[j:j + 1, :] * bcT[0:N_STATE_, j:j + 1]
            hs_ref[j * N_STATE_:(j + 1) * N_STATE_, :] = h
        h_ref[:] = h
        cdiag = jnp.where(bdiag, jnp.tile(bc8[:, N_STATE_:], (1, G_)), 0.0)
        y8 = jnp.dot(cdiag, hs_ref[:], preferred_element_type=jnp.float32)
        out_ref[0, pl.ds(base, G_), :] = (y8 + x8 * d_vec) * z8
        return 0

    jax.lax.fori_loop(0, T2 // G_, group, 0)


def _k3(g_ref, w_ref, o_ref):
    o_ref[0] = jnp.dot(g_ref[0].astype(jnp.bfloat16), w_ref[...],
                       preferred_element_type=jnp.float32)


def kernel(x, in_proj_w, conv_w, conv_b, x_proj_w, dt_proj_w, dt_proj_b,
           A_log, D, out_proj_w):
    B, L, _ = x.shape
    in_w_bf = in_proj_w.astype(jnp.bfloat16)
    out_w_bf = out_proj_w.astype(jnp.bfloat16)
    cb2 = conv_b.reshape(1, D_INNER_)
    dtb2 = dt_proj_b.reshape(1, D_INNER_)
    a_t = jnp.transpose(A_log)
    d2 = D.reshape(1, D_INNER_)

    lt1 = L // T1
    xcz, dlt, bc = pl.pallas_call(
        _k1,
        grid=(B, lt1, 4),
        in_specs=[
            pl.BlockSpec((1, T1, D_MODEL_), lambda b, l, n: (b, l, 0)),
            pl.BlockSpec((D_MODEL_, NB), lambda b, l, n: (0, n)),
            pl.BlockSpec((NB, 4), lambda b, l, n: (n % 2, 0)),
            pl.BlockSpec((1, NB), lambda b, l, n: (0, n % 2)),
            pl.BlockSpec((D_INNER_, DT_RANK_ + 2 * N_STATE_),
                         lambda b, l, n: (0, 0)),
            pl.BlockSpec((DT_RANK_, D_INNER_), lambda b, l, n: (0, 0)),
            pl.BlockSpec((1, D_INNER_), lambda b, l, n: (0, 0)),
        ],
        out_specs=[
            pl.BlockSpec((1, T1, NB), lambda b, l, n: (b, l, n)),
            pl.BlockSpec((1, T1, D_INNER_), lambda b, l, n: (b, l, 0)),
            pl.BlockSpec((1, T1, 2 * N_STATE_), lambda b, l, n: (b, l, 0)),
        ],
        out_shape=[
            jax.ShapeDtypeStruct((B, L, 2 * D_INNER_), jnp.float32),
            jax.ShapeDtypeStruct((B, L, D_INNER_), jnp.float32),
            jax.ShapeDtypeStruct((B, L, 2 * N_STATE_), jnp.float32),
        ],
        scratch_shapes=[
            pltpu.VMEM((2, 8, NB), jnp.float32),
            pltpu.VMEM((T1, D_INNER_), jnp.float32),
        ],
        compiler_params=pltpu.CompilerParams(
            dimension_semantics=("parallel", "arbitrary", "arbitrary"),
            vmem_limit_bytes=52 * 1024 * 1024,
        ),
        name="ssm_pre",
    )(x, in_w_bf, conv_w, cb2, x_proj_w, dt_proj_w, dtb2)

    dt_blocks = D_INNER_ // DB
    lt2 = L // T2
    g = pl.pallas_call(
        _k2,
        grid=(B, dt_blocks, lt2),
        in_specs=[
            pl.BlockSpec((1, T2, DB), lambda b, d, l: (b, l, d)),
            pl.BlockSpec((1, T2, DB), lambda b, d, l: (b, l, d)),
            pl.BlockSpec((1, T2, DB), lambda b, d, l: (b, l, d + D_INNER_ // DB)),
            pl.BlockSpec((1, T2, 2 * N_STATE_), lambda b, d, l: (b, l, 0)),
            pl.BlockSpec((N_STATE_, DB), lambda b, d, l: (0, d)),
            pl.BlockSpec((1, DB), lambda b, d, l: (0, d)),
        ],
        out_specs=pl.BlockSpec((1, T2, DB), lambda b, d, l: (b, l, d)),
        out_shape=jax.ShapeDtypeStruct((B, L, D_INNER_), jnp.float32),
        scratch_shapes=[pltpu.VMEM((N_STATE_, DB), jnp.float32),
                        pltpu.VMEM((G_ * N_STATE_, DB), jnp.float32)],
        compiler_params=pltpu.CompilerParams(
            dimension_semantics=("parallel", "parallel", "arbitrary"),
            vmem_limit_bytes=52 * 1024 * 1024,
        ),
        name="ssm_scan",
    )(dlt, xcz, xcz, bc, a_t, d2)

    lt3 = L // T3
    out = pl.pallas_call(
        _k3,
        grid=(B, lt3),
        in_specs=[
            pl.BlockSpec((1, T3, D_INNER_), lambda b, l: (b, l, 0)),
            pl.BlockSpec((D_INNER_, D_MODEL_), lambda b, l: (0, 0)),
        ],
        out_specs=pl.BlockSpec((1, T3, D_MODEL_), lambda b, l: (b, l, 0)),
        out_shape=jax.ShapeDtypeStruct((B, L, D_MODEL_), jnp.float32),
        compiler_params=pltpu.CompilerParams(
            dimension_semantics=("parallel", "arbitrary"),
            vmem_limit_bytes=52 * 1024 * 1024,
        ),
        name="ssm_out",
    )(g, out_w_bf)
    return out


# speedup vs baseline: 1.6321x; 1.0500x over previous
import jax
import jax.numpy as jnp
from jax.experimental import pallas as pl
from jax.experimental.pallas import tpu as pltpu

D_MODEL_ = 1024
D_INNER_ = 2048
N_STATE_ = 16
DT_RANK_ = 64
T1 = 512
NB = 1024
T3 = 512
G_ = 64

_LOG2E = 1.4426950408889634
_LN2 = 0.6931471805599453


def _silu(v):
    return v / (1.0 + jnp.exp2(v * (-_LOG2E)))


def _softplus(v):
    t = jnp.exp2(jnp.abs(v) * (-_LOG2E))
    return jnp.maximum(v, 0.0) + jnp.log2(1.0 + t) * _LN2


def _k12(x_ref, w_ref, cw_ref, cb_ref, xp_ref, dtw_ref, dtb_ref, at_ref,
         d_ref, g_ref, tail_ref, xc_s, z_s, dlt_s, bc_s, h_ref, hs_ref):
    l = pl.program_id(1)
    n = pl.program_id(2)
    H = T1 // 2

    def _taps(ext):
        return (ext[0:H] * cw_ref[:, 0] + ext[1:H + 1] * cw_ref[:, 1]
                + ext[2:H + 2] * cw_ref[:, 2] + ext[3:H + 3] * cw_ref[:, 3]
                + cb_ref[...])

    def _proj_halves():
        xh = x_ref[0].astype(jnp.bfloat16)
        xz1 = jnp.dot(xh[:H], w_ref[...], preferred_element_type=jnp.float32)
        xz2 = jnp.dot(xh[H:], w_ref[...], preferred_element_type=jnp.float32)
        return xz1, xz2

    @pl.when(n < 2)
    def _conv_half():
        xz1, xz2 = _proj_halves()
        prev = tail_ref[n, 5:8, :]
        prev = jnp.where(l == 0, jnp.zeros_like(prev), prev)
        s1 = _silu(_taps(jnp.concatenate([prev, xz1], axis=0)))
        s2 = _silu(_taps(jnp.concatenate([xz1[H - 3:], xz2], axis=0)))
        tail_ref[n] = xz2[H - 8:, :]

        @pl.when(n == 0)
        def _():
            xc_s[:H, :NB] = s1
            xc_s[H:, :NB] = s2

        @pl.when(n == 1)
        def _():
            xc_s[:H, NB:] = s1
            xc_s[H:, NB:] = s2
            x_dbl = jnp.dot(xc_s[:].astype(jnp.bfloat16),
                            xp_ref[...].astype(jnp.bfloat16),
                            preferred_element_type=jnp.float32)
            dlt_lin = jnp.dot(x_dbl[:, :DT_RANK_].astype(jnp.bfloat16),
                              dtw_ref[...].astype(jnp.bfloat16),
                              preferred_element_type=jnp.float32) + dtb_ref[...]
            dlt_s[:] = _softplus(dlt_lin)
            bc_s[:] = x_dbl[:, DT_RANK_:DT_RANK_ + 2 * N_STATE_]

    @pl.when(n == 2)
    def _gate_lo():
        xz1, xz2 = _proj_halves()
        z_s[:H, :NB] = _silu(xz1)
        z_s[H:, :NB] = _silu(xz2)

    @pl.when(n == 3)
    def _gate_hi():
        xz1, xz2 = _proj_halves()
        z_s[:H, NB:] = _silu(xz1)
        z_s[H:, NB:] = _silu(xz2)

    @pl.when(n == 4)
    def _scan():
        @pl.when(l == 0)
        def _():
            h_ref[:] = jnp.zeros_like(h_ref)

        a_mat = jnp.exp(at_ref[:]) * (-_LOG2E)
        d_vec = d_ref[:]
        lane_i = jax.lax.broadcasted_iota(jnp.int32, (G_, G_ * N_STATE_), 1)
        row_i = jax.lax.broadcasted_iota(jnp.int32, (G_, G_ * N_STATE_), 0)
        bdiag = (lane_i // N_STATE_) == row_i

        def group(i, _):
            base = pl.multiple_of(i * G_, 8)
            dg = dlt_s[pl.ds(base, G_), :]
            xg = xc_s[pl.ds(base, G_), :]
            zg = z_s[pl.ds(base, G_), :]
            bcg = bc_s[pl.ds(base, G_), :]
            bcT = jnp.transpose(bcg)
            ug = dg * xg
            h = h_ref[:]
            for j in range(G_):
                dA = jnp.exp2(dg[j:j + 1, :] * a_mat)
                h = dA * h + ug[j:j + 1, :] * bcT[0:N_STATE_, j:j + 1]
                hs_ref[j * N_STATE_:(j + 1) * N_STATE_, :] = h
            h_ref[:] = h
            cdiag = jnp.where(bdiag, jnp.tile(bcg[:, N_STATE_:], (1, G_)), 0.0)
            yg = jnp.dot(cdiag, hs_ref[:], preferred_element_type=jnp.float32)
            g_ref[0, pl.ds(base, G_), :] = (yg + xg * d_vec) * zg
            return 0

        jax.lax.fori_loop(0, T1 // G_, group, 0)


def _k3(g_ref, w_ref, o_ref):
    o_ref[0] = jnp.dot(g_ref[0].astype(jnp.bfloat16), w_ref[...],
                       preferred_element_type=jnp.float32)


def kernel(x, in_proj_w, conv_w, conv_b, x_proj_w, dt_proj_w, dt_proj_b,
           A_log, D, out_proj_w):
    B, L, _ = x.shape
    in_w_bf = in_proj_w.astype(jnp.bfloat16)
    out_w_bf = out_proj_w.astype(jnp.bfloat16)
    cb2 = conv_b.reshape(1, D_INNER_)
    dtb2 = dt_proj_b.reshape(1, D_INNER_)
    a_t = jnp.transpose(A_log)
    d2 = D.reshape(1, D_INNER_)

    lt1 = L // T1
    g = pl.pallas_call(
        _k12,
        grid=(B, lt1, 5),
        in_specs=[
            pl.BlockSpec((1, T1, D_MODEL_), lambda b, l, n: (b, l, 0)),
            pl.BlockSpec((D_MODEL_, NB), lambda b, l, n: (0, n % 4)),
            pl.BlockSpec((NB, 4), lambda b, l, n: (n % 2, 0)),
            pl.BlockSpec((1, NB), lambda b, l, n: (0, n % 2)),
            pl.BlockSpec((D_INNER_, DT_RANK_ + 2 * N_STATE_),
                         lambda b, l, n: (0, 0)),
            pl.BlockSpec((DT_RANK_, D_INNER_), lambda b, l, n: (0, 0)),
            pl.BlockSpec((1, D_INNER_), lambda b, l, n: (0, 0)),
            pl.BlockSpec((N_STATE_, D_INNER_), lambda b, l, n: (0, 0)),
            pl.BlockSpec((1, D_INNER_), lambda b, l, n: (0, 0)),
        ],
        out_specs=pl.BlockSpec((1, T1, D_INNER_), lambda b, l, n: (b, l, 0)),
        out_shape=jax.ShapeDtypeStruct((B, L, D_INNER_), jnp.float32),
        scratch_shapes=[
            pltpu.VMEM((2, 8, NB), jnp.float32),
            pltpu.VMEM((T1, D_INNER_), jnp.float32),
            pltpu.VMEM((T1, D_INNER_), jnp.float32),
            pltpu.VMEM((T1, D_INNER_), jnp.float32),
            pltpu.VMEM((T1, 2 * N_STATE_), jnp.float32),
            pltpu.VMEM((N_STATE_, D_INNER_), jnp.float32),
            pltpu.VMEM((G_ * N_STATE_, D_INNER_), jnp.float32),
        ],
        compiler_params=pltpu.CompilerParams(
            dimension_semantics=("parallel", "arbitrary", "arbitrary"),
            vmem_limit_bytes=52 * 1024 * 1024,
        ),
        name="ssm_main",
    )(x, in_w_bf, conv_w, cb2, x_proj_w, dt_proj_w, dtb2, a_t, d2)

    lt3 = L // T3
    out = pl.pallas_call(
        _k3,
        grid=(B, lt3),
        in_specs=[
            pl.BlockSpec((1, T3, D_INNER_), lambda b, l: (b, l, 0)),
            pl.BlockSpec((D_INNER_, D_MODEL_), lambda b, l: (0, 0)),
        ],
        out_specs=pl.BlockSpec((1, T3, D_MODEL_), lambda b, l: (b, l, 0)),
        out_shape=jax.ShapeDtypeStruct((B, L, D_MODEL_), jnp.float32),
        compiler_params=pltpu.CompilerParams(
            dimension_semantics=("parallel", "arbitrary"),
            vmem_limit_bytes=52 * 1024 * 1024,
        ),
        name="ssm_out",
    )(g, out_w_bf)
    return out
